# baseline (speedup 1.0000x reference)
import functools

import jax
import jax.numpy as jnp
from jax import lax
from jax.experimental import pallas as pl
from jax.experimental.pallas import tpu as pltpu
from jax.experimental.pallas import tpu_sc as plsc

N_ROWS = 32
N_COLS = 100000
K = 8
INV_T = 0.5
L = 16
U = 5
STEP = U * L
TB = 128
CH0 = 3840
CH = 32000
NCH = 3
MAIN = CH0 + NCH * CH
ALIGNED_END = MAIN + TB
NP_COLS = ALIGNED_END + TB
CPL = 256
CAND = CPL * L
BLK = 128
BIG_NEG = -3.0e38
I32_MAX = 2**31 - 1

_mesh = plsc.VectorSubcoreMesh(core_axis_name="c", subcore_axis_name="s")

NBLK = NP_COLS // BLK


def _place_body(bases_ref, blocks_ref, o_ref):
    i = pl.program_id(0)
    o_ref[...] = jnp.zeros_like(o_ref)
    for j in range(K):
        bi = bases_ref[i, j]
        o_ref[0, pl.ds(bi, 1), :] = blocks_ref[0, j, :][None, :]


_place = pl.pallas_call(
    _place_body,
    out_shape=jax.ShapeDtypeStruct((N_ROWS, NBLK, BLK), jnp.float32),
    grid=(N_ROWS,),
    in_specs=[
        pl.BlockSpec(memory_space=pltpu.SMEM),
        pl.BlockSpec((1, K, BLK), lambda i: (i, 0, 0)),
    ],
    out_specs=pl.BlockSpec((1, NBLK, BLK), lambda i: (i, 0, 0)),
)


@functools.partial(
    pl.kernel,
    mesh=_mesh,
    compiler_params=pltpu.CompilerParams(needs_layout_passes=False),
    out_type=(
        jax.ShapeDtypeStruct((N_ROWS, K * BLK), jnp.float32),
        jax.ShapeDtypeStruct((N_ROWS, BLK), jnp.int32),
    ),
    scratch_types=[
        pltpu.VMEM((N_COLS,), jnp.float32),
        pltpu.VMEM((CAND,), jnp.float32),
        pltpu.VMEM((CAND,), jnp.int32),
        pltpu.VMEM((K * BLK,), jnp.float32),
        pltpu.VMEM((BLK,), jnp.int32),
        pltpu.VMEM((TB,), jnp.float32),
        pltpu.VMEM((L,), jnp.int32),
        pltpu.SemaphoreType.DMA((NCH + 3,)),
        pltpu.SemaphoreType.DMA,
    ],
)
def _diff_topk_rows(
    cls_hbm, flat_hbm, outw_hbm, outi_hbm,
    row, cand_v, cand_i, wblk, ibuf, tbuf, posbuf, sems_in, sem_b,
):
    cid = lax.axis_index("c")
    sid = lax.axis_index("s")
    wid = sid * 2 + cid

    row_hbm = cls_hbm.at[wid]

    in0 = pltpu.async_copy(
        row_hbm.at[pl.ds(0, CH0)], row.at[pl.ds(0, CH0)], sems_in.at[0]
    )
    in_copies = [
        pltpu.async_copy(
            row_hbm.at[pl.ds(CH0 + c * CH, CH)],
            row.at[pl.ds(CH0 + c * CH, CH)],
            sems_in.at[c + 1],
        )
        for c in range(NCH)
    ]
    in_t0 = pltpu.async_copy(
        row_hbm.at[pl.ds(MAIN, TB)], row.at[pl.ds(MAIN, TB)],
        sems_in.at[NCH + 1],
    )
    tail_off = wid * N_COLS + ALIGNED_END
    p = lax.rem(tail_off, TB)
    in_t1 = pltpu.async_copy(
        flat_hbm.at[pl.ds(pl.multiple_of(tail_off - p, TB), TB)],
        tbuf,
        sems_in.at[NCH + 2],
    )

    lanes = lax.iota(jnp.int32, L)
    neg = jnp.full((L,), BIG_NEG, jnp.float32)
    imax_v = jnp.full((L,), I32_MAX, jnp.int32)
    zf = jnp.zeros((L,), jnp.float32)

    def init_cand(i, c):
        cand_v[pl.ds(i * L, L)] = neg
        cand_i[pl.ds(i * L, L)] = imax_v
        return c

    lax.fori_loop(0, CAND // L, init_cand, 0)

    def init_w(i, c):
        wblk[pl.ds(i * L, L)] = zf
        return c

    lax.fori_loop(0, K * BLK // L, init_w, 0)

    in0.wait()

    def pre_body(i, carry):
        m0, m1 = carry
        base = i * STEP
        for u in range(U):
            v = row[pl.ds(base + u * L, L)]
            if u % 2 == 0:
                m0 = jnp.maximum(m0, v)
            else:
                m1 = jnp.maximum(m1, v)
        return (m0, m1)

    m0, m1 = lax.fori_loop(0, CH0 // STEP, pre_body, (neg, neg))
    mv = jnp.maximum(m0, m1)
    for _ in range(K - 1):
        cur = jnp.max(mv)
        mv = jnp.where(mv == cur, neg, mv)
    thr = jnp.max(mv)

    lim = CAND - L + lanes
    posbuf[pl.ds(0, L)] = lanes

    def scan_body(i, carry, _off=0):
        a0, a1 = carry
        base = _off + i * STEP
        vs = []
        msks = []
        for u in range(U):
            v = row[pl.ds(base + u * L, L)]
            e = jnp.exp(v * INV_T)
            if u % 2 == 0:
                a0 = a0 + e
            else:
                a1 = a1 + e
            vs.append(v)
            msks.append(v >= thr)
        anym = msks[0]
        for u in range(1, U):
            anym = anym | msks[u]

        @pl.when(jnp.max(jnp.where(anym, 1, 0)) > 0)
        def _collect():
            posv = posbuf[pl.ds(0, L)]
            for u in range(U):
                off = base + u * L
                pos = jnp.minimum(posv, lim)
                plsc.store_scatter(cand_v, [pos], vs[u], mask=msks[u])
                plsc.store_scatter(cand_i, [pos], off + lanes, mask=msks[u])
                posv = posv + jnp.where(msks[u], L, 0)
            posbuf[pl.ds(0, L)] = posv

        return (a0, a1)

    carry = lax.fori_loop(
        0, CH0 // STEP, functools.partial(scan_body, _off=0), (zf, zf)
    )
    for c in range(NCH):
        in_copies[c].wait()
        carry = lax.fori_loop(
            0,
            CH // STEP,
            functools.partial(scan_body, _off=CH0 + c * CH),
            carry,
        )
    in_t0.wait()
    in_t1.wait()
    for u in range(2):
        row[pl.ds(ALIGNED_END + u * L, L)] = tbuf[pl.ds(p + u * L, L)]
    carry = lax.fori_loop(
        0,
        (N_COLS - MAIN) // STEP,
        functools.partial(scan_body, _off=MAIN),
        carry,
    )
    a0, a1 = carry
    s_total = jnp.sum(a0 + a1)
    posv = posbuf[pl.ds(0, L)]
    n_slots = jnp.max(posv - lanes) // L

    chosen_v = []
    chosen_i = []
    for j in range(K):

        def sel_body(c, carry, _chosen_i=tuple(chosen_i)):
            bv, bi = carry
            v = cand_v[pl.ds(c * L, L)]
            ii = cand_i[pl.ds(c * L, L)]
            better = (v > bv) | ((v == bv) & (ii < bi))
            for pj in _chosen_i:
                better = better & (ii != pj)
            bv = jnp.where(better, v, bv)
            bi = jnp.where(better, ii, bi)
            return (bv, bi)

        bv, bi = lax.fori_loop(0, n_slots, sel_body, (neg, imax_v))
        vj = jnp.max(bv)
        ij = jnp.min(jnp.where(bv == vj, bi, imax_v))
        chosen_v.append(vj)
        chosen_i.append(ij)

    v_vec = neg
    i_vec = jnp.zeros((L,), jnp.int32)
    for j in range(K):
        sel = lanes == j
        v_vec = jnp.where(sel, chosen_v[j], v_vec)
        i_vec = jnp.where(sel, chosen_i[j], i_vec)
    e_vec = jnp.exp(v_vec * INV_T)
    d_vec = s_total - (plsc.cumsum(e_vec) - e_vec)
    w_vec = e_vec / d_vec

    valid = lanes < K
    bvec = jnp.zeros((L,), jnp.int32)
    for j in range(K):
        base_j = jnp.bitwise_and(chosen_i[j], -BLK)
        local = i_vec - base_j
        msk = valid & (local >= 0) & (local < BLK)
        idx = j * BLK + jnp.minimum(jnp.maximum(local, 0), BLK - 1)
        plsc.store_scatter(wblk, [idx], w_vec, mask=msk)
        bvec = jnp.where(lanes == j, base_j // BLK, bvec)
    for b in range(BLK // L):
        ibuf[pl.ds(b * L, L)] = jnp.zeros((L,), jnp.int32)
    ibuf[pl.ds(0, L)] = bvec
    cp_w = pltpu.async_copy(wblk, outw_hbm.at[wid], sem_b)
    cp_i = pltpu.async_copy(ibuf, outi_hbm.at[wid], sem_b)
    cp_w.wait()
    cp_i.wait()


def kernel(normu, cls):
    wblks, bidx = _diff_topk_rows(cls, cls.reshape(-1))
    placed = _place(bidx, wblks.reshape(N_ROWS, K, BLK))
    return (normu, placed.reshape(N_ROWS, NP_COLS)[:, :N_COLS])


# speedup vs baseline: 1.5660x; 1.5660x over previous
import functools

import jax
import jax.numpy as jnp
from jax import lax
from jax.experimental import pallas as pl
from jax.experimental.pallas import tpu as pltpu
from jax.experimental.pallas import tpu_sc as plsc

N_ROWS = 32
N_COLS = 100000
K = 8
INV_T = 0.5
L = 16
U = 10
STEP = U * L
TB = 128
CH0 = 3840
CH = 32000
NCH = 3
MAIN = CH0 + NCH * CH
ALIGNED_END = MAIN + TB
NP_COLS = ALIGNED_END + TB
ZN = 12800
NZ7 = 7
ZREM = NP_COLS - NZ7 * ZN
CPL = 256
CAND = CPL * L
BLK = 128
BIG_NEG = -3.0e38
I32_MAX = 2**31 - 1

_mesh = plsc.VectorSubcoreMesh(core_axis_name="c", subcore_axis_name="s")


@functools.partial(
    pl.kernel,
    mesh=_mesh,
    compiler_params=pltpu.CompilerParams(needs_layout_passes=False),
    out_type=jax.ShapeDtypeStruct((N_ROWS, NP_COLS), jnp.float32),
    scratch_types=[
        pltpu.VMEM((N_COLS,), jnp.float32),
        pltpu.VMEM((ZN,), jnp.float32),
        pltpu.VMEM((CAND,), jnp.float32),
        pltpu.VMEM((CAND,), jnp.int32),
        pltpu.VMEM((K * BLK,), jnp.float32),
        pltpu.VMEM((TB,), jnp.float32),
        pltpu.VMEM((L,), jnp.int32),
        pltpu.SemaphoreType.DMA((NCH + 3,)),
        pltpu.SemaphoreType.DMA,
        pltpu.SemaphoreType.DMA,
    ],
)
def _diff_topk_rows(
    cls_hbm, flat_hbm, out_hbm,
    row, zbuf, cand_v, cand_i, wblk, tbuf, posbuf, sems_in, sem_z, sem_b,
):
    cid = lax.axis_index("c")
    sid = lax.axis_index("s")
    wid = sid * 2 + cid

    row_hbm = cls_hbm.at[wid]
    orow_hbm = out_hbm.at[wid]

    in0 = pltpu.async_copy(
        row_hbm.at[pl.ds(0, CH0)], row.at[pl.ds(0, CH0)], sems_in.at[0]
    )
    in_copies = [
        pltpu.async_copy(
            row_hbm.at[pl.ds(CH0 + c * CH, CH)],
            row.at[pl.ds(CH0 + c * CH, CH)],
            sems_in.at[c + 1],
        )
        for c in range(NCH)
    ]
    in_t0 = pltpu.async_copy(
        row_hbm.at[pl.ds(MAIN, TB)], row.at[pl.ds(MAIN, TB)],
        sems_in.at[NCH + 1],
    )
    tail_off = wid * N_COLS + ALIGNED_END
    p = lax.rem(tail_off, TB)
    in_t1 = pltpu.async_copy(
        flat_hbm.at[pl.ds(pl.multiple_of(tail_off - p, TB), TB)],
        tbuf,
        sems_in.at[NCH + 2],
    )

    lanes = lax.iota(jnp.int32, L)
    neg = jnp.full((L,), BIG_NEG, jnp.float32)
    imax_v = jnp.full((L,), I32_MAX, jnp.int32)
    zf = jnp.zeros((L,), jnp.float32)

    def init_cand(i, c):
        cand_v[pl.ds(i * L, L)] = neg
        cand_i[pl.ds(i * L, L)] = imax_v
        return c

    lax.fori_loop(0, CAND // L, init_cand, 0)

    def init_z(i, c):
        base = i * STEP
        for u in range(U):
            zbuf[pl.ds(base + u * L, L)] = zf
        return c

    lax.fori_loop(0, ZN // STEP, init_z, 0)

    def init_w(i, c):
        wblk[pl.ds(i * L, L)] = zf
        return c

    lax.fori_loop(0, K * BLK // L, init_w, 0)

    z_copies = [
        pltpu.async_copy(zbuf, orow_hbm.at[pl.ds(z * ZN, ZN)], sem_z)
        for z in range(NZ7)
    ]
    z_copies.append(
        pltpu.async_copy(
            zbuf.at[pl.ds(0, ZREM)],
            orow_hbm.at[pl.ds(NZ7 * ZN, ZREM)],
            sem_z,
        )
    )

    in0.wait()

    def pre_body(i, carry):
        m0, m1 = carry
        base = i * STEP
        for u in range(U):
            v = row[pl.ds(base + u * L, L)]
            if u % 2 == 0:
                m0 = jnp.maximum(m0, v)
            else:
                m1 = jnp.maximum(m1, v)
        return (m0, m1)

    m0, m1 = lax.fori_loop(0, CH0 // STEP, pre_body, (neg, neg))
    mv = jnp.maximum(m0, m1)
    for _ in range(K - 1):
        cur = jnp.max(mv)
        mv = jnp.where(mv == cur, neg, mv)
    thr = jnp.max(mv)

    lim = CAND - L + lanes
    posbuf[pl.ds(0, L)] = lanes

    def scan_body(i, carry, _off=0):
        a0, a1 = carry
        base = _off + i * STEP
        vs = []
        msks = []
        for u in range(U):
            v = row[pl.ds(base + u * L, L)]
            e = jnp.exp(v * INV_T)
            if u % 2 == 0:
                a0 = a0 + e
            else:
                a1 = a1 + e
            vs.append(v)
            msks.append(v >= thr)
        anym = msks[0]
        for u in range(1, U):
            anym = anym | msks[u]

        @pl.when(jnp.max(jnp.where(anym, 1, 0)) > 0)
        def _collect():
            posv = posbuf[pl.ds(0, L)]
            for u in range(U):
                off = base + u * L
                pos = jnp.minimum(posv, lim)
                plsc.store_scatter(cand_v, [pos], vs[u], mask=msks[u])
                plsc.store_scatter(cand_i, [pos], off + lanes, mask=msks[u])
                posv = posv + jnp.where(msks[u], L, 0)
            posbuf[pl.ds(0, L)] = posv

        return (a0, a1)

    carry = lax.fori_loop(
        0, CH0 // STEP, functools.partial(scan_body, _off=0), (zf, zf)
    )
    for c in range(NCH):
        in_copies[c].wait()
        carry = lax.fori_loop(
            0,
            CH // STEP,
            functools.partial(scan_body, _off=CH0 + c * CH),
            carry,
        )
    in_t0.wait()
    in_t1.wait()
    for u in range(2):
        row[pl.ds(ALIGNED_END + u * L, L)] = tbuf[pl.ds(p + u * L, L)]
    carry = lax.fori_loop(
        0,
        (N_COLS - MAIN) // STEP,
        functools.partial(scan_body, _off=MAIN),
        carry,
    )
    a0, a1 = carry
    s_total = jnp.sum(a0 + a1)
    posv = posbuf[pl.ds(0, L)]
    n_slots = jnp.max(posv - lanes) // L

    chosen_v = []
    chosen_i = []
    for j in range(K):

        def sel_body(c, carry, _chosen_i=tuple(chosen_i)):
            bv, bi = carry
            v = cand_v[pl.ds(c * L, L)]
            ii = cand_i[pl.ds(c * L, L)]
            better = (v > bv) | ((v == bv) & (ii < bi))
            for pj in _chosen_i:
                better = better & (ii != pj)
            bv = jnp.where(better, v, bv)
            bi = jnp.where(better, ii, bi)
            return (bv, bi)

        bv, bi = lax.fori_loop(0, n_slots, sel_body, (neg, imax_v))
        vj = jnp.max(bv)
        ij = jnp.min(jnp.where(bv == vj, bi, imax_v))
        chosen_v.append(vj)
        chosen_i.append(ij)

    v_vec = neg
    i_vec = jnp.zeros((L,), jnp.int32)
    for j in range(K):
        sel = lanes == j
        v_vec = jnp.where(sel, chosen_v[j], v_vec)
        i_vec = jnp.where(sel, chosen_i[j], i_vec)
    e_vec = jnp.exp(v_vec * INV_T)
    d_vec = s_total - (plsc.cumsum(e_vec) - e_vec)
    w_vec = e_vec / d_vec

    for z in z_copies:
        z.wait()
    valid = lanes < K
    b_copies = []
    for j in range(K):
        base_j = jnp.bitwise_and(chosen_i[j], -BLK)
        local = i_vec - base_j
        msk = valid & (local >= 0) & (local < BLK)
        idx = j * BLK + jnp.minimum(jnp.maximum(local, 0), BLK - 1)
        plsc.store_scatter(wblk, [idx], w_vec, mask=msk)
        b_copies.append(
            pltpu.async_copy(
                wblk.at[pl.ds(j * BLK, BLK)],
                orow_hbm.at[pl.ds(pl.multiple_of(base_j, BLK), BLK)],
                sem_b,
            )
        )
    for b in b_copies:
        b.wait()


def kernel(normu, cls):
    padded = _diff_topk_rows(cls, cls.reshape(-1))
    return (normu, padded[:, :N_COLS])


# speedup vs baseline: 1.5818x; 1.0101x over previous
import functools

import jax
import jax.numpy as jnp
from jax import lax
from jax.experimental import pallas as pl
from jax.experimental.pallas import tpu as pltpu
from jax.experimental.pallas import tpu_sc as plsc

N_ROWS = 32
N_COLS = 100000
K = 8
INV_T = 0.5
L = 16
U = 10
STEP = U * L
TB = 128
CH0 = 3840
CH = 32000
NCH = 3
MAIN = CH0 + NCH * CH
ALIGNED_END = MAIN + TB
NP_COLS = ALIGNED_END + TB
ZN = 12800
NZ7 = 7
ZREM = NP_COLS - NZ7 * ZN
CPL = 256
CAND = CPL * L
BLK = 128
BIG_NEG = -3.0e38
I32_MAX = 2**31 - 1

_mesh = plsc.VectorSubcoreMesh(core_axis_name="c", subcore_axis_name="s")


@functools.partial(
    pl.kernel,
    mesh=_mesh,
    compiler_params=pltpu.CompilerParams(needs_layout_passes=False),
    out_type=jax.ShapeDtypeStruct((N_ROWS, NP_COLS), jnp.float32),
    scratch_types=[
        pltpu.VMEM((N_COLS,), jnp.float32),
        pltpu.VMEM((ZN,), jnp.float32),
        pltpu.VMEM((CAND,), jnp.float32),
        pltpu.VMEM((CAND,), jnp.int32),
        pltpu.VMEM((K * BLK,), jnp.float32),
        pltpu.VMEM((TB,), jnp.float32),
        pltpu.VMEM((L,), jnp.int32),
        pltpu.SemaphoreType.DMA((NCH + 3,)),
        pltpu.SemaphoreType.DMA,
        pltpu.SemaphoreType.DMA,
    ],
)
def _diff_topk_rows(
    cls_hbm, flat_hbm, out_hbm,
    row, zbuf, cand_v, cand_i, wblk, tbuf, posbuf, sems_in, sem_z, sem_b,
):
    cid = lax.axis_index("c")
    sid = lax.axis_index("s")
    wid = sid * 2 + cid

    row_hbm = cls_hbm.at[wid]
    orow_hbm = out_hbm.at[wid]

    in0 = pltpu.async_copy(
        row_hbm.at[pl.ds(0, CH0)], row.at[pl.ds(0, CH0)], sems_in.at[0]
    )
    in_copies = [
        pltpu.async_copy(
            row_hbm.at[pl.ds(CH0 + c * CH, CH)],
            row.at[pl.ds(CH0 + c * CH, CH)],
            sems_in.at[c + 1],
        )
        for c in range(NCH)
    ]
    in_t0 = pltpu.async_copy(
        row_hbm.at[pl.ds(MAIN, TB)], row.at[pl.ds(MAIN, TB)],
        sems_in.at[NCH + 1],
    )
    tail_off = wid * N_COLS + ALIGNED_END
    p = lax.rem(tail_off, TB)
    in_t1 = pltpu.async_copy(
        flat_hbm.at[pl.ds(pl.multiple_of(tail_off - p, TB), TB)],
        tbuf,
        sems_in.at[NCH + 2],
    )

    lanes = lax.iota(jnp.int32, L)
    neg = jnp.full((L,), BIG_NEG, jnp.float32)
    imax_v = jnp.full((L,), I32_MAX, jnp.int32)
    zf = jnp.zeros((L,), jnp.float32)

    def init_cand(i, c):
        cand_v[pl.ds(i * L, L)] = neg
        cand_i[pl.ds(i * L, L)] = imax_v
        return c

    lax.fori_loop(0, CAND // L, init_cand, 0)

    def init_z(i, c):
        base = i * STEP
        for u in range(U):
            zbuf[pl.ds(base + u * L, L)] = zf
        return c

    lax.fori_loop(0, ZN // STEP, init_z, 0)

    def init_w(i, c):
        wblk[pl.ds(i * L, L)] = zf
        return c

    lax.fori_loop(0, K * BLK // L, init_w, 0)

    z_copies = [
        pltpu.async_copy(zbuf, orow_hbm.at[pl.ds(z * ZN, ZN)], sem_z)
        for z in range(NZ7)
    ]
    z_copies.append(
        pltpu.async_copy(
            zbuf.at[pl.ds(0, ZREM)],
            orow_hbm.at[pl.ds(NZ7 * ZN, ZREM)],
            sem_z,
        )
    )

    in0.wait()

    def pre_body(i, carry):
        m0, m1 = carry
        base = i * STEP
        for u in range(U):
            v = row[pl.ds(base + u * L, L)]
            if u % 2 == 0:
                m0 = jnp.maximum(m0, v)
            else:
                m1 = jnp.maximum(m1, v)
        return (m0, m1)

    m0, m1 = lax.fori_loop(0, CH0 // STEP, pre_body, (neg, neg))
    mv = jnp.maximum(m0, m1)
    for _ in range(K - 1):
        cur = jnp.max(mv)
        mv = jnp.where(mv == cur, neg, mv)
    thr = jnp.max(mv)

    lim = CAND - L + lanes
    posbuf[pl.ds(0, L)] = lanes

    def scan_body(i, carry, _off=0):
        acc = list(carry)
        base = _off + i * STEP
        vs = []
        msks = []
        for u in range(U):
            v = row[pl.ds(base + u * L, L)]
            e = jnp.exp(v * INV_T)
            acc[u % 4] = acc[u % 4] + e
            vs.append(v)
            msks.append(v >= thr)
        anym = msks[0]
        for u in range(1, U):
            anym = anym | msks[u]

        @pl.when(jnp.max(jnp.where(anym, 1, 0)) > 0)
        def _collect():
            posv = posbuf[pl.ds(0, L)]
            for u in range(U):
                off = base + u * L
                pos = jnp.minimum(posv, lim)
                plsc.store_scatter(cand_v, [pos], vs[u], mask=msks[u])
                plsc.store_scatter(cand_i, [pos], off + lanes, mask=msks[u])
                posv = posv + jnp.where(msks[u], L, 0)
            posbuf[pl.ds(0, L)] = posv

        return tuple(acc)

    carry = lax.fori_loop(
        0, CH0 // STEP, functools.partial(scan_body, _off=0), (zf, zf, zf, zf)
    )
    for c in range(NCH):
        in_copies[c].wait()
        carry = lax.fori_loop(
            0,
            CH // STEP,
            functools.partial(scan_body, _off=CH0 + c * CH),
            carry,
        )
    in_t0.wait()
    in_t1.wait()
    for u in range(2):
        row[pl.ds(ALIGNED_END + u * L, L)] = tbuf[pl.ds(p + u * L, L)]
    carry = lax.fori_loop(
        0,
        (N_COLS - MAIN) // STEP,
        functools.partial(scan_body, _off=MAIN),
        carry,
    )
    a0, a1, a2, a3 = carry
    s_total = jnp.sum((a0 + a1) + (a2 + a3))
    posv = posbuf[pl.ds(0, L)]
    n_slots = jnp.max(posv - lanes) // L

    chosen_v = []
    chosen_i = []
    for j in range(K):

        def sel_body(c, carry, _chosen_i=tuple(chosen_i)):
            bv, bi = carry
            v = cand_v[pl.ds(c * L, L)]
            ii = cand_i[pl.ds(c * L, L)]
            better = (v > bv) | ((v == bv) & (ii < bi))
            for pj in _chosen_i:
                better = better & (ii != pj)
            bv = jnp.where(better, v, bv)
            bi = jnp.where(better, ii, bi)
            return (bv, bi)

        bv, bi = lax.fori_loop(0, n_slots, sel_body, (neg, imax_v))
        vj = jnp.max(bv)
        ij = jnp.min(jnp.where(bv == vj, bi, imax_v))
        chosen_v.append(vj)
        chosen_i.append(ij)

    v_vec = neg
    i_vec = jnp.zeros((L,), jnp.int32)
    for j in range(K):
        sel = lanes == j
        v_vec = jnp.where(sel, chosen_v[j], v_vec)
        i_vec = jnp.where(sel, chosen_i[j], i_vec)
    e_vec = jnp.exp(v_vec * INV_T)
    d_vec = s_total - (plsc.cumsum(e_vec) - e_vec)
    w_vec = e_vec / d_vec

    for z in z_copies:
        z.wait()
    valid = lanes < K
    b_copies = []
    for j in range(K):
        base_j = jnp.bitwise_and(chosen_i[j], -BLK)
        local = i_vec - base_j
        msk = valid & (local >= 0) & (local < BLK)
        idx = j * BLK + jnp.minimum(jnp.maximum(local, 0), BLK - 1)
        plsc.store_scatter(wblk, [idx], w_vec, mask=msk)
        b_copies.append(
            pltpu.async_copy(
                wblk.at[pl.ds(j * BLK, BLK)],
                orow_hbm.at[pl.ds(pl.multiple_of(base_j, BLK), BLK)],
                sem_b,
            )
        )
    for b in b_copies:
        b.wait()


def kernel(normu, cls):
    padded = _diff_topk_rows(cls, cls.reshape(-1))
    return (normu, padded[:, :N_COLS])
